# baseline (speedup 1.0000x reference)
import functools

import jax
import jax.numpy as jnp
from jax import lax
from jax.experimental import pallas as pl
from jax.experimental.pallas import tpu as pltpu
from jax.experimental.pallas import tpu_sc as plsc

_NC = 2
_NS = 16
_NW = _NC * _NS
_LANES = 16


@functools.cache
def _build_pairtype(E, A, n):
    per_w = E // _NW
    assert per_w * _NW == E and per_w % 8 == 0
    n_groups = -(-per_w // _LANES)
    n_full = per_w // _LANES
    buf = n_groups * _LANES
    unroll = 8
    assert n_full % unroll == 0

    mesh = plsc.VectorSubcoreMesh(core_axis_name="c", subcore_axis_name="s")

    @functools.partial(
        pl.kernel,
        out_type=jax.ShapeDtypeStruct((E,), jnp.int32),
        mesh=mesh,
        compiler_params=pltpu.CompilerParams(needs_layout_passes=False),
        scratch_types=[
            pltpu.VMEM((buf,), jnp.int32),
            pltpu.VMEM((buf,), jnp.int32),
            pltpu.VMEM((buf,), jnp.int32),
            pltpu.VMEM((A,), jnp.int32),
            pltpu.VMEM((n * n,), jnp.int32),
        ],
    )
    def sc_call(z1_hbm, z2_hbm, aw_hbm, pi_hbm, pt_hbm, z1v, z2v, ptv, aw_v, pi_v):
        wid = lax.axis_index("s") * _NC + lax.axis_index("c")
        base = wid * per_w

        ones_i = jnp.ones((_LANES,), jnp.int32)
        z1v[pl.ds(n_full * _LANES, _LANES)] = ones_i
        z2v[pl.ds(n_full * _LANES, _LANES)] = ones_i

        pltpu.sync_copy(aw_hbm, aw_v)
        pltpu.sync_copy(pi_hbm, pi_v)
        pltpu.sync_copy(z1_hbm.at[pl.ds(base, per_w)], z1v.at[pl.ds(0, per_w)])
        pltpu.sync_copy(z2_hbm.at[pl.ds(base, per_w)], z2v.at[pl.ds(0, per_w)])

        def group(g):
            ia = plsc.load_gather(aw_v, [z1v[pl.ds(g * _LANES, _LANES)] - 1])
            ib = plsc.load_gather(aw_v, [z2v[pl.ds(g * _LANES, _LANES)] - 1])
            ptv[pl.ds(g * _LANES, _LANES)] = plsc.load_gather(pi_v, [ia * n + ib])

        def loop_body(t, carry):
            for j in range(unroll):
                group(t * unroll + j)
            return carry
        lax.fori_loop(0, n_full // unroll, loop_body, 0)
        for g in range(n_full, n_groups):
            group(g)

        pltpu.sync_copy(ptv.at[pl.ds(0, per_w)], pt_hbm.at[pl.ds(base, per_w)])

    return sc_call


@functools.cache
def _build_onehot_t(E, C):
    assert E % 128 == 0
    rows = E // 128
    pt_rows_pb = 128
    cols_pb = pt_rows_pb * 128
    grid = -(-rows // pt_rows_pb)

    def body(pt_ref, out_ref):
        ciota = lax.broadcasted_iota(jnp.int32, (C, 128), 0)
        for j in range(pt_rows_pb):
            row = pt_ref[pl.ds(j, 1), :]
            out_ref[:, pl.ds(j * 128, 128)] = (row == ciota).astype(jnp.float32)

    return pl.pallas_call(
        body,
        grid=(grid,),
        in_specs=[pl.BlockSpec((pt_rows_pb, 128), lambda i: (i, 0))],
        out_specs=pl.BlockSpec((C, cols_pb), lambda i: (0, i)),
        out_shape=jax.ShapeDtypeStruct((C, E), jnp.float32),
    )


def kernel(z1, z2, atom_weight, pair_ids, onehot_table):
    E = z1.shape[0]
    A = atom_weight.shape[0]
    n = pair_ids.shape[0]
    C = onehot_table.shape[1]
    pt = _build_pairtype(E, A, n)(z1, z2, atom_weight, pair_ids.reshape(-1))
    return _build_onehot_t(E, C)(pt.reshape(E // 128, 128)).T


# speedup vs baseline: 1.0213x; 1.0213x over previous
import functools

import jax
import jax.numpy as jnp
from jax import lax
from jax.experimental import pallas as pl
from jax.experimental.pallas import tpu as pltpu
from jax.experimental.pallas import tpu_sc as plsc

_NC = 2
_NS = 16
_NW = _NC * _NS
_LANES = 16


@functools.cache
def _build_pairtype(E, A, n):
    per_w = E // _NW
    assert per_w * _NW == E and per_w % 8 == 0
    n_groups = -(-per_w // _LANES)
    n_full = per_w // _LANES
    buf = n_groups * _LANES
    unroll = 8
    assert n_full % unroll == 0

    mesh = plsc.VectorSubcoreMesh(core_axis_name="c", subcore_axis_name="s")

    @functools.partial(
        pl.kernel,
        out_type=jax.ShapeDtypeStruct((E,), jnp.int32),
        mesh=mesh,
        compiler_params=pltpu.CompilerParams(needs_layout_passes=False),
        scratch_types=[
            pltpu.VMEM((buf,), jnp.int32),
            pltpu.VMEM((buf,), jnp.int32),
            pltpu.VMEM((buf,), jnp.int32),
            pltpu.VMEM((A,), jnp.int32),
            pltpu.VMEM((n * n,), jnp.int32),
        ],
    )
    def sc_call(z1_hbm, z2_hbm, aw_hbm, pi_hbm, pt_hbm, z1v, z2v, ptv, aw_v, pi_v):
        wid = lax.axis_index("s") * _NC + lax.axis_index("c")
        base = wid * per_w

        ones_i = jnp.ones((_LANES,), jnp.int32)
        z1v[pl.ds(n_full * _LANES, _LANES)] = ones_i
        z2v[pl.ds(n_full * _LANES, _LANES)] = ones_i

        pltpu.sync_copy(aw_hbm, aw_v)
        pltpu.sync_copy(pi_hbm, pi_v)
        pltpu.sync_copy(z1_hbm.at[pl.ds(base, per_w)], z1v.at[pl.ds(0, per_w)])
        pltpu.sync_copy(z2_hbm.at[pl.ds(base, per_w)], z2v.at[pl.ds(0, per_w)])

        def group(g):
            ia = plsc.load_gather(aw_v, [z1v[pl.ds(g * _LANES, _LANES)] - 1])
            ib = plsc.load_gather(aw_v, [z2v[pl.ds(g * _LANES, _LANES)] - 1])
            ptv[pl.ds(g * _LANES, _LANES)] = plsc.load_gather(pi_v, [ia * n + ib])

        def loop_body(t, carry):
            for j in range(unroll):
                group(t * unroll + j)
            return carry
        lax.fori_loop(0, n_full // unroll, loop_body, 0)
        for g in range(n_full, n_groups):
            group(g)

        pltpu.sync_copy(ptv.at[pl.ds(0, per_w)], pt_hbm.at[pl.ds(base, per_w)])

    return sc_call


@functools.cache
def _build_onehot_t(E, C):
    assert E % 128 == 0
    rows = E // 128
    pt_rows_pb = 64
    cols_pb = pt_rows_pb * 128
    grid = -(-rows // pt_rows_pb)

    def body(pt_ref, out_ref):
        ciota = lax.broadcasted_iota(jnp.int32, (C, 128), 0)
        for j in range(pt_rows_pb):
            row = pt_ref[pl.ds(j, 1), :]
            out_ref[:, pl.ds(j * 128, 128)] = (row == ciota).astype(jnp.float32)

    return pl.pallas_call(
        body,
        grid=(grid,),
        in_specs=[pl.BlockSpec((pt_rows_pb, 128), lambda i: (i, 0))],
        out_specs=pl.BlockSpec((C, cols_pb), lambda i: (0, i)),
        out_shape=jax.ShapeDtypeStruct((C, E), jnp.float32),
    )


def kernel(z1, z2, atom_weight, pair_ids, onehot_table):
    E = z1.shape[0]
    A = atom_weight.shape[0]
    n = pair_ids.shape[0]
    C = onehot_table.shape[1]
    pt = _build_pairtype(E, A, n)(z1, z2, atom_weight, pair_ids.reshape(-1))
    return _build_onehot_t(E, C)(pt.reshape(E // 128, 128)).T
